# baseline (speedup 1.0000x reference)
import functools

import jax
import jax.numpy as jnp
from jax import lax
from jax.experimental import pallas as pl
from jax.experimental.pallas import tpu as pltpu
from jax.experimental.pallas import tpu_sc as plsc

_VOCAB = 100000
_HIDDEN = 32
_BATCH = 1024

_info = plsc.get_sparse_core_info()
_NC, _NS = _info.num_cores, _info.num_subcores
_NW = _NC * _NS
_B_PER_W = _BATCH // _NW

_sc_mesh = plsc.VectorSubcoreMesh(core_axis_name="c", subcore_axis_name="s")


@functools.partial(
    pl.kernel,
    mesh=_sc_mesh,
    out_type=jax.ShapeDtypeStruct((_BATCH, _HIDDEN), jnp.float32),
    scratch_types=[
        pltpu.VMEM((_B_PER_W,), jnp.int32),
        pltpu.VMEM((_B_PER_W, _HIDDEN), jnp.float32),
        pltpu.SemaphoreType.DMA,
    ],
    compiler_params=pltpu.CompilerParams(use_tc_tiling_on_sc=False),
)
def _sc_gather(idx_hbm, table_hbm, out_hbm, idx_v, rows_v, sem):
    wid = lax.axis_index("s") * _NC + lax.axis_index("c")
    base = wid * _B_PER_W
    pltpu.sync_copy(idx_hbm.at[pl.ds(base, _B_PER_W)], idx_v)
    pltpu.async_copy(table_hbm.at[idx_v], rows_v, sem).wait()
    pltpu.sync_copy(rows_v, out_hbm.at[pl.ds(base, _B_PER_W)])


_RB = 16
_GRID = _BATCH // _RB


def _proj_body(h_ref, w_ref, b_ref, out_ref):
    acc = lax.dot_general(
        h_ref[...], w_ref[...],
        (((1,), (1,)), ((), ())),
        preferred_element_type=jnp.float32,
    )
    out_ref[...] = acc + b_ref[...]


_proj = pl.pallas_call(
    _proj_body,
    grid=(_GRID,),
    in_specs=[
        pl.BlockSpec((_RB, _HIDDEN), lambda i: (i, 0)),
        pl.BlockSpec((_VOCAB, _HIDDEN), lambda i: (0, 0)),
        pl.BlockSpec((1, _VOCAB), lambda i: (0, 0)),
    ],
    out_specs=pl.BlockSpec((_RB, _VOCAB), lambda i: (i, 0)),
    out_shape=jax.ShapeDtypeStruct((_BATCH, _VOCAB), jnp.float32),
    compiler_params=pltpu.CompilerParams(vmem_limit_bytes=100 * 1024 * 1024),
)


def kernel(input_ids, embed_table, proj_weight, proj_bias):
    hidden = jnp.take(embed_table, input_ids, axis=0)
    return _proj(hidden, proj_weight, proj_bias.reshape(1, _VOCAB))


# speedup vs baseline: 1.5216x; 1.5216x over previous
import functools

import jax
import jax.numpy as jnp
from jax import lax
from jax.experimental import pallas as pl
from jax.experimental.pallas import tpu as pltpu
from jax.experimental.pallas import tpu_sc as plsc

_VOCAB = 100000
_HIDDEN = 32
_BATCH = 1024

_info = plsc.get_sparse_core_info()
_NC, _NS = _info.num_cores, _info.num_subcores
_NW = _NC * _NS
_B_PER_W = _BATCH // _NW

_sc_mesh = plsc.VectorSubcoreMesh(core_axis_name="c", subcore_axis_name="s")


@functools.partial(
    pl.kernel,
    mesh=_sc_mesh,
    out_type=jax.ShapeDtypeStruct((_BATCH, _HIDDEN), jnp.float32),
    scratch_types=[
        pltpu.VMEM((_B_PER_W,), jnp.int32),
        pltpu.VMEM((_B_PER_W, _HIDDEN), jnp.float32),
        pltpu.SemaphoreType.DMA,
    ],
    compiler_params=pltpu.CompilerParams(use_tc_tiling_on_sc=False),
)
def _sc_gather(idx_hbm, table_hbm, out_hbm, idx_v, rows_v, sem):
    wid = lax.axis_index("s") * _NC + lax.axis_index("c")
    base = wid * _B_PER_W
    pltpu.sync_copy(idx_hbm.at[pl.ds(base, _B_PER_W)], idx_v)
    pltpu.async_copy(table_hbm.at[idx_v], rows_v, sem).wait()
    pltpu.sync_copy(rows_v, out_hbm.at[pl.ds(base, _B_PER_W)])


_VB = 1024
_NBUF = 6
_GRID = pl.cdiv(_VOCAB, _VB)
_TAIL = _VOCAB - (_GRID - 1) * _VB


def _proj_body(h_ref, w_ref, b_ref, out_ref, acc_ref, tail_ref, sem):
    i = pl.program_id(0)
    slot = lax.rem(i, _NBUF)

    @pl.when(i >= _NBUF)
    def _():
        pltpu.make_async_copy(
            acc_ref.at[slot],
            out_ref.at[:, pl.ds((i - _NBUF) * _VB, _VB)],
            sem.at[slot],
        ).wait()

    acc = lax.dot_general(
        h_ref[...], w_ref[...],
        (((1,), (1,)), ((), ())),
        preferred_element_type=jnp.float32,
    )
    acc_ref[slot] = acc + b_ref[...]

    @pl.when(i < _GRID - 1)
    def _():
        pltpu.make_async_copy(
            acc_ref.at[slot],
            out_ref.at[:, pl.ds(i * _VB, _VB)],
            sem.at[slot],
        ).start()

    @pl.when(i == _GRID - 1)
    def _():
        tail_ref[...] = acc_ref[slot, :, : _TAIL]
        pltpu.make_async_copy(
            tail_ref,
            out_ref.at[:, pl.ds((_GRID - 1) * _VB, _TAIL)],
            sem.at[slot],
        ).start()
        for j in range(_GRID - _NBUF, _GRID - 1):
            pltpu.make_async_copy(
                acc_ref.at[j % _NBUF],
                out_ref.at[:, pl.ds(j * _VB, _VB)],
                sem.at[j % _NBUF],
            ).wait()
        pltpu.make_async_copy(
            tail_ref,
            out_ref.at[:, pl.ds((_GRID - 1) * _VB, _TAIL)],
            sem.at[(_GRID - 1) % _NBUF],
        ).wait()


_proj = pl.pallas_call(
    _proj_body,
    grid=(_GRID,),
    in_specs=[
        pl.BlockSpec((_BATCH, _HIDDEN), lambda i: (0, 0)),
        pl.BlockSpec((_VB, _HIDDEN), lambda i: (i, 0)),
        pl.BlockSpec((1, _VB), lambda i: (0, i)),
    ],
    out_specs=pl.BlockSpec(memory_space=pl.ANY),
    out_shape=jax.ShapeDtypeStruct((_BATCH, _VOCAB), jnp.float32),
    scratch_shapes=[
        pltpu.VMEM((_NBUF, _BATCH, _VB), jnp.float32),
        pltpu.VMEM((_BATCH, _TAIL), jnp.float32),
        pltpu.SemaphoreType.DMA((_NBUF,)),
    ],
    compiler_params=pltpu.CompilerParams(vmem_limit_bytes=100 * 1024 * 1024),
)


def kernel(input_ids, embed_table, proj_weight, proj_bias):
    hidden = jnp.take(embed_table, input_ids, axis=0)
    return _proj(hidden, proj_weight, proj_bias.reshape(1, _VOCAB))


# speedup vs baseline: 1.6373x; 1.0760x over previous
import functools

import jax
import jax.numpy as jnp
from jax import lax
from jax.experimental import pallas as pl
from jax.experimental.pallas import tpu as pltpu
from jax.experimental.pallas import tpu_sc as plsc

_VOCAB = 100000
_HIDDEN = 32
_BATCH = 1024

_info = plsc.get_sparse_core_info()
_NC, _NS = _info.num_cores, _info.num_subcores
_NW = _NC * _NS
_B_PER_W = _BATCH // _NW

_sc_mesh = plsc.VectorSubcoreMesh(core_axis_name="c", subcore_axis_name="s")


@functools.partial(
    pl.kernel,
    mesh=_sc_mesh,
    out_type=jax.ShapeDtypeStruct((_BATCH, _HIDDEN), jnp.float32),
    scratch_types=[
        pltpu.VMEM((_B_PER_W,), jnp.int32),
        pltpu.VMEM((_B_PER_W, _HIDDEN), jnp.float32),
        pltpu.SemaphoreType.DMA,
    ],
    compiler_params=pltpu.CompilerParams(use_tc_tiling_on_sc=False),
)
def _sc_gather(idx_hbm, table_hbm, out_hbm, idx_v, rows_v, sem):
    wid = lax.axis_index("s") * _NC + lax.axis_index("c")
    base = wid * _B_PER_W
    pltpu.sync_copy(idx_hbm.at[pl.ds(base, _B_PER_W)], idx_v)
    pltpu.async_copy(table_hbm.at[idx_v], rows_v, sem).wait()
    pltpu.sync_copy(rows_v, out_hbm.at[pl.ds(base, _B_PER_W)])


_RB = 64
_GRID = _BATCH // _RB


def _proj_body(h_ref, w_ref, b_ref, out_ref):
    acc = lax.dot_general(
        h_ref[...], w_ref[...],
        (((1,), (0,)), ((), ())),
        preferred_element_type=jnp.float32,
    )
    out_ref[...] = acc + b_ref[...]


_proj = pl.pallas_call(
    _proj_body,
    grid=(_GRID,),
    in_specs=[
        pl.BlockSpec((_RB, _HIDDEN), lambda i: (i, 0)),
        pl.BlockSpec((_HIDDEN, _VOCAB), lambda i: (0, 0)),
        pl.BlockSpec((1, _VOCAB), lambda i: (0, 0)),
    ],
    out_specs=pl.BlockSpec((_RB, _VOCAB), lambda i: (i, 0)),
    out_shape=jax.ShapeDtypeStruct((_BATCH, _VOCAB), jnp.float32),
    compiler_params=pltpu.CompilerParams(vmem_limit_bytes=100 * 1024 * 1024),
)


def kernel(input_ids, embed_table, proj_weight, proj_bias):
    hidden = jnp.take(embed_table, input_ids, axis=0)
    return _proj(hidden, proj_weight.T, proj_bias.reshape(1, _VOCAB))
